# baseline (device time: 2039289 ns/iter reference)
import jax
import jax.numpy as jnp
from jax import lax
from jax.experimental import pallas as pl
from jax.experimental.pallas import tpu as pltpu

NCHUNK = 8


def kernel(x):
    m, n2 = x.shape
    n = n2 // 2
    rows = m // NCHUNK

    def body(x_ref, out_ref, copy_sems):
        my_x = lax.axis_index("x")
        copies = []
        for c in range(NCHUNK):
            cp = pltpu.make_async_copy(
                x_ref.at[pl.ds(c * rows, rows), pl.ds(my_x * n, n)],
                out_ref.at[pl.ds(my_x * m + c * rows, rows), :],
                copy_sems.at[c],
            )
            cp.start()
            copies.append(cp)
        for cp in copies:
            cp.wait()

    return pl.pallas_call(
        body,
        out_shape=jax.ShapeDtypeStruct((2 * m, n), jnp.float32),
        in_specs=[pl.BlockSpec(memory_space=pl.ANY)],
        out_specs=pl.BlockSpec(memory_space=pl.ANY),
        scratch_shapes=[
            pltpu.SemaphoreType.DMA((NCHUNK,)),
        ],
    )(x)


# device time: 62453 ns/iter; 32.6532x vs baseline; 32.6532x over previous
import jax
import jax.numpy as jnp
from jax import lax
from jax.experimental import pallas as pl
from jax.experimental.pallas import tpu as pltpu

NBUF = 4
ROWS = 1024


def kernel(x):
    m, n2 = x.shape
    n = n2 // 2
    nchunk = m // ROWS

    def body(x_ref, out_ref, vmem, in_sems, out_sems):
        my_x = lax.axis_index("x")
        out_cps = [None] * nchunk
        for c in range(nchunk):
            slot = c % NBUF
            if c >= NBUF:
                out_cps[c - NBUF].wait()
            in_cp = pltpu.make_async_copy(
                x_ref.at[pl.ds(c * ROWS, ROWS), pl.ds(my_x * n, n)],
                vmem.at[slot],
                in_sems.at[slot],
            )
            in_cp.start()
            in_cp.wait()
            out_cp = pltpu.make_async_copy(
                vmem.at[slot],
                out_ref.at[pl.ds(my_x * m + c * ROWS, ROWS), :],
                out_sems.at[slot],
            )
            out_cp.start()
            out_cps[c] = out_cp
        for c in range(nchunk - NBUF, nchunk):
            out_cps[c].wait()

    return pl.pallas_call(
        body,
        out_shape=jax.ShapeDtypeStruct((2 * m, n), jnp.float32),
        in_specs=[pl.BlockSpec(memory_space=pl.ANY)],
        out_specs=pl.BlockSpec(memory_space=pl.ANY),
        scratch_shapes=[
            pltpu.VMEM((NBUF, ROWS, 1024), jnp.float32),
            pltpu.SemaphoreType.DMA((NBUF,)),
            pltpu.SemaphoreType.DMA((NBUF,)),
        ],
    )(x)
